# baseline (speedup 1.0000x reference)
import jax
import jax.numpy as jnp
from jax import lax
from jax.experimental import pallas as pl
from jax.experimental.pallas import tpu as pltpu
from jax.experimental.pallas import tpu_sc as plsc

N = 1048576
B = 4096
L = 512
CLS = 101
NW = 32
C = N // NW
E = 4608
SL = E // NW
RPW = B // NW
P = 2048

_MESH = plsc.VectorSubcoreMesh(
    core_axis_name="c", subcore_axis_name="s", num_cores=2, num_subcores=16
)
_PARAMS = pltpu.CompilerParams(needs_layout_passes=False)


def _wid():
    return lax.axis_index("s") * 2 + lax.axis_index("c")


def _k1_run_ends(seg_hbm, ends_hbm, blk_hbm, segbuf, e1, blkbuf, sem):
    w = _wid()
    base = w * C
    pltpu.async_copy(seg_hbm.at[pl.ds(base, C)], segbuf.at[pl.ds(0, C)], sem)

    def _zero(i, carry):
        e1[pl.ds(i * 16, 16)] = jnp.zeros((16,), jnp.int32)
        return carry

    lax.fori_loop(0, E // 16, _zero, 0)
    pltpu.make_async_copy(seg_hbm.at[pl.ds(0, C)], segbuf.at[pl.ds(0, C)], sem).wait()

    @pl.when(w < NW - 1)
    def _load_lookahead():
        pltpu.sync_copy(seg_hbm.at[pl.ds(base + C, 16)], segbuf.at[pl.ds(C, 16)])

    @pl.when(w == NW - 1)
    def _sentinel():
        segbuf[pl.ds(C, 16)] = jnp.full((16,), B, jnp.int32)

    iota = lax.iota(jnp.int32, 16)

    def _scan(v, carry):
        bb = v * 64
        for u in range(4):
            cur = segbuf[pl.ds(bb + u * 16, 16)]
            nxt = segbuf[pl.ds(bb + u * 16 + 1, 16)]
            gi = base + bb + u * 16 + iota
            plsc.store_scatter(e1, [cur + 1], gi + 1, mask=cur != nxt)
        return carry

    lax.fori_loop(0, C // 64, _scan, 0)
    pltpu.sync_copy(e1, ends_hbm.at[pl.ds(w * E, E)])

    blkbuf[pl.ds(0, 16)] = jnp.zeros((16,), jnp.int32)
    blkbuf[pl.ds(16, 16)] = jnp.zeros((16,), jnp.int32)
    run = jnp.zeros((16,), jnp.int32)
    for b in range(1, NW):
        for j in range(RPW // 16):
            run = jnp.maximum(run, e1[pl.ds((b - 1) * RPW + j * 16, 16)])
        plsc.store_scatter(
            blkbuf, [jnp.full((16,), b, jnp.int32)], jnp.broadcast_to(jnp.max(run), (16,)),
            mask=iota == 0,
        )
    pltpu.sync_copy(blkbuf, blk_hbm.at[pl.ds(w * NW, NW)])


def _k2_emit_rows(
    tok_hbm, seg_hbm, ends_hbm, blk_hbm, out_hbm,
    wnd, sumv, starts, segp0, tokp0, segp1, tokp1, rowbuf, semc, sem0, sem1
):
    w = _wid()
    r0 = pl.multiple_of(w * RPW, 8)
    pltpu.async_copy(blk_hbm, sumv, semc)

    def _wload(u, carry):
        pltpu.async_copy(
            ends_hbm.at[pl.ds(u * E + r0, SL)], wnd.at[pl.ds(u * SL, SL)], semc
        )
        return carry

    lax.fori_loop(0, NW, _wload, 0)

    iota = lax.iota(jnp.int32, 16)
    zv = jnp.zeros((16,), jnp.int32)

    def _zero(i, carry):
        bb = i * 128
        for u in range(8):
            rowbuf[pl.ds(bb + u * 16, 16)] = zv
        return carry

    lax.fori_loop(0, RPW * L // 128, _zero, 0)
    for jj in range(RPW // 16):
        plsc.store_scatter(rowbuf, [(jj * 16 + iota) * L], jnp.full((16,), CLS, jnp.int32))

    pltpu.make_async_copy(blk_hbm, sumv, semc).wait()
    pltpu.make_async_copy(ends_hbm.at[pl.ds(0, NW * SL)], wnd, semc).wait()

    m0 = plsc.load_gather(sumv, [iota * NW + w])
    m1 = plsc.load_gather(sumv, [(iota + 16) * NW + w])
    run = jnp.max(jnp.maximum(m0, m1))

    lo = None
    hi = None
    for j in range(SL // 16):
        cv = wnd[pl.ds(j * 16, 16)]
        def _mx(u, vv):
            return jnp.maximum(vv, wnd[pl.ds(u * SL + j * 16, 16)])
        cv = lax.fori_loop(1, NW, _mx, cv)
        sv = jnp.maximum(jnp.broadcast_to(run, (16,)), plsc.cummax(cv))
        starts[pl.ds(j * 16, 16)] = sv
        lane0 = jnp.max(jnp.where(iota == 0, sv, 0))
        if j == 0:
            lo = lane0
        if j == RPW // 16:
            hi = lane0
        run = jnp.max(sv)

    lo8 = lo & ~7
    npieces = (hi - lo8 + P - 1) // P

    def _pstart(pp):
        return pl.multiple_of(jnp.minimum(lo8 + pp * P, N - P), 8)

    def _fetch(pp, sp, tp, sem):
        st = _pstart(pp)
        pltpu.async_copy(seg_hbm.at[pl.ds(st, P)], sp, sem)
        pltpu.async_copy(tok_hbm.at[pl.ds(st, P)], tp, sem)

    def _compute(p, sp, tp, sem, edge):
        pltpu.make_async_copy(seg_hbm.at[pl.ds(0, P)], sp, sem).wait()
        pltpu.make_async_copy(tok_hbm.at[pl.ds(0, P)], tp, sem).wait()
        st = _pstart(p)

        def _vec(v, inner):
            bb = v * 64
            for u in range(4):
                sg = sp[pl.ds(bb + u * 16, 16)]
                tk = tp[pl.ds(bb + u * 16, 16)]
                gi = st + bb + u * 16 + iota
                if edge:
                    inrange = (gi >= lo) & (gi < hi)
                    ss = plsc.load_gather(starts, [sg - r0], mask=inrange)
                    pos = gi - ss + 1
                    valid = inrange & (pos < L)
                    off = jnp.clip((sg - r0) * L + pos, 0, RPW * L - 1)
                else:
                    ss = plsc.load_gather(starts, [sg - r0])
                    pos = gi - ss + 1
                    valid = pos < L
                    off = jnp.clip((sg - r0) * L + pos, 0, RPW * L - 1)
                plsc.store_scatter(rowbuf, [off], tk, mask=valid)
            return inner

        lax.fori_loop(0, P // 64, _vec, 0)

    @pl.when(npieces > 0)
    def _prime():
        _fetch(0, segp0, tokp0, sem0)

    def _piece(p, carry):
        even = p % 2 == 0
        more = p + 1 < npieces
        is_edge = (p == 0) | (p == npieces - 1)

        @pl.when(more & even)
        def _next_odd():
            _fetch(p + 1, segp1, tokp1, sem1)

        @pl.when(more & (~even))
        def _next_even():
            _fetch(p + 1, segp0, tokp0, sem0)

        @pl.when(even & is_edge)
        def _do_even_edge():
            _compute(p, segp0, tokp0, sem0, True)

        @pl.when(even & (~is_edge))
        def _do_even_mid():
            _compute(p, segp0, tokp0, sem0, False)

        @pl.when((~even) & is_edge)
        def _do_odd_edge():
            _compute(p, segp1, tokp1, sem1, True)

        @pl.when((~even) & (~is_edge))
        def _do_odd_mid():
            _compute(p, segp1, tokp1, sem1, False)

        return carry

    lax.fori_loop(0, npieces, _piece, 0)
    pltpu.sync_copy(rowbuf, out_hbm.at[pl.ds(r0 * L, RPW * L)])


@jax.jit
def kernel(flat_wp_tokens, segment_ids):
    i32 = jnp.int32
    ends, blk = pl.kernel(
        _k1_run_ends,
        out_type=(
            jax.ShapeDtypeStruct((NW * E,), i32),
            jax.ShapeDtypeStruct((NW * NW,), i32),
        ),
        mesh=_MESH,
        compiler_params=_PARAMS,
        scratch_types=[
            pltpu.VMEM((C + 16,), i32),
            pltpu.VMEM((E,), i32),
            pltpu.VMEM((NW,), i32),
            pltpu.SemaphoreType.DMA,
        ],
    )(segment_ids)
    out_flat = pl.kernel(
        _k2_emit_rows,
        out_type=jax.ShapeDtypeStruct((B * L,), i32),
        mesh=_MESH,
        compiler_params=_PARAMS,
        scratch_types=[
            pltpu.VMEM((NW * SL,), i32),
            pltpu.VMEM((NW * NW,), i32),
            pltpu.VMEM((SL,), i32),
            pltpu.VMEM((P,), i32),
            pltpu.VMEM((P,), i32),
            pltpu.VMEM((P,), i32),
            pltpu.VMEM((P,), i32),
            pltpu.VMEM((RPW * L,), i32),
            pltpu.SemaphoreType.DMA,
            pltpu.SemaphoreType.DMA,
            pltpu.SemaphoreType.DMA,
        ],
    )(flat_wp_tokens, segment_ids, ends, blk)
    return out_flat.reshape(B, L)


# speedup vs baseline: 1.6521x; 1.6521x over previous
import jax
import jax.numpy as jnp
from jax import lax
from jax.experimental import pallas as pl
from jax.experimental.pallas import tpu as pltpu
from jax.experimental.pallas import tpu_sc as plsc

N = 1048576
B = 4096
L = 512
CLS = 101
NW = 32
C = N // NW
E = 4608
SL = E // NW
RPW = B // NW
P = 2048

_MESH = plsc.VectorSubcoreMesh(
    core_axis_name="c", subcore_axis_name="s", num_cores=2, num_subcores=16
)
_PARAMS = pltpu.CompilerParams(needs_layout_passes=False)


def _wid():
    return lax.axis_index("s") * 2 + lax.axis_index("c")


def _k1_run_ends(seg_hbm, ends_hbm, blk_hbm, segbuf, e1, blkbuf, sem):
    w = _wid()
    base = w * C
    pltpu.async_copy(seg_hbm.at[pl.ds(base, C)], segbuf.at[pl.ds(0, C)], sem)

    def _zero(i, carry):
        e1[pl.ds(i * 16, 16)] = jnp.zeros((16,), jnp.int32)
        return carry

    lax.fori_loop(0, E // 16, _zero, 0)
    pltpu.make_async_copy(seg_hbm.at[pl.ds(0, C)], segbuf.at[pl.ds(0, C)], sem).wait()

    @pl.when(w < NW - 1)
    def _load_lookahead():
        pltpu.sync_copy(seg_hbm.at[pl.ds(base + C, 16)], segbuf.at[pl.ds(C, 16)])

    @pl.when(w == NW - 1)
    def _sentinel():
        segbuf[pl.ds(C, 16)] = jnp.full((16,), B, jnp.int32)

    iota = lax.iota(jnp.int32, 16)

    def _scan(v):
        cur = segbuf[pl.ds(v * 16, 16)]
        nxt = segbuf[pl.ds(v * 16 + 1, 16)]
        gi = base + v * 16 + iota
        plsc.store_scatter(e1, [cur + 1], gi + 1, mask=cur != nxt)

    plsc.parallel_loop(0, C // 16, unroll=4)(_scan)
    pltpu.sync_copy(e1, ends_hbm.at[pl.ds(w * E, E)])

    blkbuf[pl.ds(0, 16)] = jnp.zeros((16,), jnp.int32)
    blkbuf[pl.ds(16, 16)] = jnp.zeros((16,), jnp.int32)
    run = jnp.zeros((16,), jnp.int32)
    for b in range(1, NW):
        for j in range(RPW // 16):
            run = jnp.maximum(run, e1[pl.ds((b - 1) * RPW + j * 16, 16)])
        plsc.store_scatter(
            blkbuf, [jnp.full((16,), b, jnp.int32)], jnp.broadcast_to(jnp.max(run), (16,)),
            mask=iota == 0,
        )
    pltpu.sync_copy(blkbuf, blk_hbm.at[pl.ds(w * NW, NW)])


def _k2_emit_rows(
    tok_hbm, seg_hbm, ends_hbm, blk_hbm, out_hbm,
    wnd, sumv, starts, segp0, tokp0, segp1, tokp1, rowbuf, semc, sem0, sem1
):
    w = _wid()
    r0 = pl.multiple_of(w * RPW, 8)
    pltpu.async_copy(blk_hbm, sumv, semc)

    def _wload(u, carry):
        pltpu.async_copy(
            ends_hbm.at[pl.ds(u * E + r0, SL)], wnd.at[pl.ds(u * SL, SL)], semc
        )
        return carry

    lax.fori_loop(0, NW, _wload, 0)

    iota = lax.iota(jnp.int32, 16)
    zv = jnp.zeros((16,), jnp.int32)

    def _zero(i):
        rowbuf[pl.ds(i * 16, 16)] = zv

    plsc.parallel_loop(0, RPW * L // 16, unroll=8)(_zero)
    for jj in range(RPW // 16):
        plsc.store_scatter(rowbuf, [(jj * 16 + iota) * L], jnp.full((16,), CLS, jnp.int32))

    pltpu.make_async_copy(blk_hbm, sumv, semc).wait()
    pltpu.make_async_copy(ends_hbm.at[pl.ds(0, NW * SL)], wnd, semc).wait()

    m0 = plsc.load_gather(sumv, [iota * NW + w])
    m1 = plsc.load_gather(sumv, [(iota + 16) * NW + w])
    run = jnp.max(jnp.maximum(m0, m1))

    lo = None
    hi = None
    for j in range(SL // 16):
        cv = wnd[pl.ds(j * 16, 16)]
        def _mx(u, vv):
            return jnp.maximum(vv, wnd[pl.ds(u * SL + j * 16, 16)])
        cv = lax.fori_loop(1, NW, _mx, cv)
        sv = jnp.maximum(jnp.broadcast_to(run, (16,)), plsc.cummax(cv))
        starts[pl.ds(j * 16, 16)] = sv
        lane0 = jnp.max(jnp.where(iota == 0, sv, 0))
        if j == 0:
            lo = lane0
        if j == RPW // 16:
            hi = lane0
        run = jnp.max(sv)

    lo8 = lo & ~7
    npieces = (hi - lo8 + P - 1) // P

    def _pstart(pp):
        return pl.multiple_of(jnp.minimum(lo8 + pp * P, N - P), 8)

    def _fetch(pp, sp, tp, sem):
        st = _pstart(pp)
        pltpu.async_copy(seg_hbm.at[pl.ds(st, P)], sp, sem)
        pltpu.async_copy(tok_hbm.at[pl.ds(st, P)], tp, sem)

    def _compute(p, sp, tp, sem, edge):
        pltpu.make_async_copy(seg_hbm.at[pl.ds(0, P)], sp, sem).wait()
        pltpu.make_async_copy(tok_hbm.at[pl.ds(0, P)], tp, sem).wait()
        st = _pstart(p)

        def _vec(v):
            sg = sp[pl.ds(v * 16, 16)]
            tk = tp[pl.ds(v * 16, 16)]
            gi = st + v * 16 + iota
            if edge:
                inrange = (gi >= lo) & (gi < hi)
                ss = plsc.load_gather(starts, [sg - r0], mask=inrange)
                pos = gi - ss + 1
                valid = inrange & (pos < L)
                off = jnp.clip((sg - r0) * L + pos, 0, RPW * L - 1)
            else:
                ss = plsc.load_gather(starts, [sg - r0])
                pos = gi - ss + 1
                valid = pos < L
                off = jnp.clip((sg - r0) * L + pos, 0, RPW * L - 1)
            plsc.store_scatter(rowbuf, [off], tk, mask=valid)

        plsc.parallel_loop(0, P // 16, unroll=4)(_vec)

    @pl.when(npieces > 0)
    def _prime():
        _fetch(0, segp0, tokp0, sem0)

    def _piece(p, carry):
        even = p % 2 == 0
        more = p + 1 < npieces
        is_edge = (p == 0) | (p == npieces - 1)

        @pl.when(more & even)
        def _next_odd():
            _fetch(p + 1, segp1, tokp1, sem1)

        @pl.when(more & (~even))
        def _next_even():
            _fetch(p + 1, segp0, tokp0, sem0)

        @pl.when(even & is_edge)
        def _do_even_edge():
            _compute(p, segp0, tokp0, sem0, True)

        @pl.when(even & (~is_edge))
        def _do_even_mid():
            _compute(p, segp0, tokp0, sem0, False)

        @pl.when((~even) & is_edge)
        def _do_odd_edge():
            _compute(p, segp1, tokp1, sem1, True)

        @pl.when((~even) & (~is_edge))
        def _do_odd_mid():
            _compute(p, segp1, tokp1, sem1, False)

        return carry

    lax.fori_loop(0, npieces, _piece, 0)
    pltpu.sync_copy(rowbuf, out_hbm.at[pl.ds(r0 * L, RPW * L)])


@jax.jit
def kernel(flat_wp_tokens, segment_ids):
    i32 = jnp.int32
    ends, blk = pl.kernel(
        _k1_run_ends,
        out_type=(
            jax.ShapeDtypeStruct((NW * E,), i32),
            jax.ShapeDtypeStruct((NW * NW,), i32),
        ),
        mesh=_MESH,
        compiler_params=_PARAMS,
        scratch_types=[
            pltpu.VMEM((C + 16,), i32),
            pltpu.VMEM((E,), i32),
            pltpu.VMEM((NW,), i32),
            pltpu.SemaphoreType.DMA,
        ],
    )(segment_ids)
    out_flat = pl.kernel(
        _k2_emit_rows,
        out_type=jax.ShapeDtypeStruct((B * L,), i32),
        mesh=_MESH,
        compiler_params=_PARAMS,
        scratch_types=[
            pltpu.VMEM((NW * SL,), i32),
            pltpu.VMEM((NW * NW,), i32),
            pltpu.VMEM((SL,), i32),
            pltpu.VMEM((P,), i32),
            pltpu.VMEM((P,), i32),
            pltpu.VMEM((P,), i32),
            pltpu.VMEM((P,), i32),
            pltpu.VMEM((RPW * L,), i32),
            pltpu.SemaphoreType.DMA,
            pltpu.SemaphoreType.DMA,
            pltpu.SemaphoreType.DMA,
        ],
    )(flat_wp_tokens, segment_ids, ends, blk)
    return out_flat.reshape(B, L)
